# baseline (speedup 1.0000x reference)
import functools

import jax
import jax.numpy as jnp
from jax import lax
from jax.experimental import pallas as pl
from jax.experimental.pallas import tpu as pltpu
from jax.experimental.pallas import tpu_sc as plsc

A, BS, T, V = 8, 4096, 50, 32
NROWS = A * T
NC, NS = 2, 16
NW = NC * NS
SUBR = 1024

TC_ROWS = 136
SC_ROWS = NROWS - TC_ROWS
SR_PER_W = SC_ROWS * 4 // NW
PER_W = SUBR * SR_PER_W
SR_PER_CH = 11
CH = SUBR * SR_PER_CH
N_CHUNKS = SR_PER_W // SR_PER_CH
JV = SUBR // 16

_mesh = plsc.VectorSubcoreMesh(core_axis_name="c", subcore_axis_name="s")


@functools.partial(
    pl.kernel,
    mesh=_mesh,
    out_type=jax.ShapeDtypeStruct((NW, SR_PER_W * 16), jnp.float32),
    compiler_params=pltpu.CompilerParams(needs_layout_passes=False),
    scratch_types=[
        pltpu.VMEM((CH,), jnp.int32),
        pltpu.VMEM((CH,), jnp.int32),
        pltpu.VMEM((CH,), jnp.float32),
        pltpu.VMEM((CH,), jnp.float32),
        pltpu.VMEM((CH,), jnp.int32),
        pltpu.VMEM((CH,), jnp.int32),
        pltpu.VMEM((CH,), jnp.float32),
        pltpu.VMEM((CH,), jnp.float32),
        pltpu.VMEM((SR_PER_W * 16,), jnp.float32),
        pltpu.SemaphoreType.DMA,
        pltpu.SemaphoreType.DMA,
    ],
)
def _sc_gather_reduce(lp_hbm, act_hbm, td_hbm, out_hbm,
                      act0, act1, td0, td1, idx0, idx1, gat0, gat1,
                      acc_v, sem_in, sem_g):
    c = lax.axis_index("c")
    s = lax.axis_index("s")
    wid = s * NC + c
    pbase = wid * PER_W
    g0 = wid * SR_PER_W
    lanes = lax.iota(jnp.int32, 16)
    act_b, td_b, idx_b, gat_b = (act0, act1), (td0, td1), (idx0, idx1), (gat0, gat1)

    def load_and_index(ci, b):
        cb = pbase + ci * CH
        pltpu.async_copy(act_hbm.at[pl.ds(cb, CH)], act_b[b], sem_in)
        pltpu.async_copy(td_hbm.at[pl.ds(cb, CH)], td_b[b], sem_in).wait()
        pltpu.make_async_copy(act_hbm.at[pl.ds(cb, CH)], act_b[b],
                              sem_in).wait()

        def sub_body(r, _):
            g = g0 + ci * SR_PER_CH + r
            plane = (g >> 2) * (V * BS)
            b0 = (g & 3) << 10

            def idx_body(j, _):
                bj = b0 + j * 16
                sb = plane + ((bj >> 7) << 10) + (bj & 127)
                av = act_b[b][pl.ds(r * SUBR + j * 16, 16)]
                idx_b[b][pl.ds(r * SUBR + j * 16, 16)] = (
                    (sb + lanes) + ((av >> 3) << 15) + ((av & 7) << 7))
                return 0
            lax.fori_loop(0, JV, idx_body, 0, unroll=8)
            return 0
        lax.fori_loop(0, SR_PER_CH, sub_body, 0)

    def fire(b):
        pltpu.async_copy(lp_hbm.at[idx_b[b]], gat_b[b], sem_g)

    def drain(b):
        pltpu.make_async_copy(lp_hbm.at[idx_b[b]], gat_b[b], sem_g).wait()

    def accumulate(ci, b):
        def sub_body(r, _):
            def acc_body(j, av):
                d = r * SUBR + j * 16
                return av + gat_b[b][pl.ds(d, 16)] * td_b[b][pl.ds(d, 16)]
            av = lax.fori_loop(0, JV, acc_body, jnp.zeros((16,), jnp.float32),
                               unroll=8)
            acc_v[pl.ds((ci * SR_PER_CH + r) * 16, 16)] = av
            return 0
        lax.fori_loop(0, SR_PER_CH, sub_body, 0)

    load_and_index(0, 0)
    fire(0)
    for ci in range(1, N_CHUNKS):
        b = ci % 2
        load_and_index(ci, b)
        drain(1 - b)
        fire(b)
        accumulate(ci - 1, 1 - b)
    drain((N_CHUNKS - 1) % 2)
    accumulate(N_CHUNKS - 1, (N_CHUNKS - 1) % 2)
    pltpu.sync_copy(acc_v, out_hbm.at[wid])


def _tc_body(lp_ref, act_ref, td_ref, out_ref):
    act = act_ref[0]
    td = td_ref[0]
    acc = jnp.zeros((32, 128), jnp.float32)
    for v4 in range(4):
        for v8 in range(8):
            acc = acc + jnp.where(act == v4 * 8 + v8,
                                  lp_ref[0, v4, :, v8, :], 0.0)
    out_ref[0] = acc * td


_tc_select = pl.pallas_call(
    _tc_body,
    grid=(TC_ROWS,),
    in_specs=[
        pl.BlockSpec((1, 4, 32, 8, 128), lambda r: (SC_ROWS + r, 0, 0, 0, 0)),
        pl.BlockSpec((1, 32, 128), lambda r: (SC_ROWS + r, 0, 0)),
        pl.BlockSpec((1, 32, 128), lambda r: (SC_ROWS + r, 0, 0)),
    ],
    out_specs=pl.BlockSpec((1, 32, 128), lambda r: (r, 0, 0)),
    out_shape=jax.ShapeDtypeStruct((TC_ROWS, 32, 128), jnp.float32),
)


def kernel(log_policies, td_errors, actions):
    lp6 = jnp.transpose(
        jnp.transpose(log_policies, (0, 2, 3, 1))
        .reshape(A, T, V // 8, 8, BS // 128, 128),
        (0, 1, 2, 4, 3, 5))
    lp_flat = lp6.reshape(-1)
    lp_rows = lp6.reshape(NROWS, V // 8, BS // 128, 8, 128)
    act_flat = jnp.transpose(actions.astype(jnp.int32), (0, 2, 3, 1)).reshape(-1)
    td_flat = jnp.transpose(td_errors.astype(jnp.float32), (0, 2, 3, 1)).reshape(-1)
    act_rows = act_flat.reshape(NROWS, BS // 128, 128)
    td_rows = td_flat.reshape(NROWS, BS // 128, 128)

    sc_partials = _sc_gather_reduce(lp_flat, act_flat, td_flat)
    tc_partials = _tc_select(lp_rows, act_rows, td_rows)

    sc_rows = sc_partials.reshape(SC_ROWS, 4 * 16).sum(axis=-1)
    tc_rows = tc_partials.reshape(TC_ROWS, 32 * 128).sum(axis=-1)
    rows = jnp.concatenate([sc_rows, tc_rows])
    return rows.reshape(A, T) * (-1.0 / BS)


# speedup vs baseline: 1.4801x; 1.4801x over previous
import functools

import jax
import jax.numpy as jnp
from jax import lax
from jax.experimental import pallas as pl
from jax.experimental.pallas import tpu as pltpu
from jax.experimental.pallas import tpu_sc as plsc

A, BS, T, V = 8, 4096, 50, 32
NROWS = A * T
NC, NS = 2, 16
NW = NC * NS
SUBR = 1024

TC_ROWS = 80
SC_ROWS = NROWS - TC_ROWS
SR_PER_W = SC_ROWS * 4 // NW
PER_W = SUBR * SR_PER_W
SR_PER_CH = 10
CH = SUBR * SR_PER_CH
N_CHUNKS = SR_PER_W // SR_PER_CH
TC_BLK = 2
JV = SUBR // 16

_mesh = plsc.VectorSubcoreMesh(core_axis_name="c", subcore_axis_name="s")


@functools.partial(
    pl.kernel,
    mesh=_mesh,
    out_type=jax.ShapeDtypeStruct((NW, SR_PER_W * 16), jnp.float32),
    compiler_params=pltpu.CompilerParams(needs_layout_passes=False),
    scratch_types=[
        pltpu.VMEM((CH,), jnp.int32),
        pltpu.VMEM((CH,), jnp.int32),
        pltpu.VMEM((CH,), jnp.float32),
        pltpu.VMEM((CH,), jnp.float32),
        pltpu.VMEM((CH,), jnp.int32),
        pltpu.VMEM((CH,), jnp.int32),
        pltpu.VMEM((CH,), jnp.float32),
        pltpu.VMEM((CH,), jnp.float32),
        pltpu.VMEM((SR_PER_W * 16,), jnp.float32),
        pltpu.SemaphoreType.DMA,
        pltpu.SemaphoreType.DMA,
    ],
)
def _sc_gather_reduce(lp_hbm, act_hbm, td_hbm, out_hbm,
                      act0, act1, td0, td1, idx0, idx1, gat0, gat1,
                      acc_v, sem_in, sem_g):
    c = lax.axis_index("c")
    s = lax.axis_index("s")
    wid = s * NC + c
    pbase = wid * PER_W
    g0 = wid * SR_PER_W
    lanes = lax.iota(jnp.int32, 16)
    act_b, td_b, idx_b, gat_b = (act0, act1), (td0, td1), (idx0, idx1), (gat0, gat1)

    def load_and_index(ci, b):
        cb = pbase + ci * CH
        pltpu.async_copy(act_hbm.at[pl.ds(cb, CH)], act_b[b], sem_in)
        pltpu.async_copy(td_hbm.at[pl.ds(cb, CH)], td_b[b], sem_in).wait()
        pltpu.make_async_copy(act_hbm.at[pl.ds(cb, CH)], act_b[b],
                              sem_in).wait()

        def sub_body(r, _):
            g = g0 + ci * SR_PER_CH + r
            plane = (g >> 2) * (V * BS)
            b0 = (g & 3) << 10

            def idx_body(j, _):
                bj = b0 + j * 16
                sb = plane + ((bj >> 7) << 10) + (bj & 127)
                av = act_b[b][pl.ds(r * SUBR + j * 16, 16)]
                idx_b[b][pl.ds(r * SUBR + j * 16, 16)] = (
                    (sb + lanes) + ((av >> 3) << 15) + ((av & 7) << 7))
                return 0
            lax.fori_loop(0, JV, idx_body, 0, unroll=8)
            return 0
        lax.fori_loop(0, SR_PER_CH, sub_body, 0)

    def fire(b):
        pltpu.async_copy(lp_hbm.at[idx_b[b]], gat_b[b], sem_g)

    def drain(b):
        pltpu.make_async_copy(lp_hbm.at[idx_b[b]], gat_b[b], sem_g).wait()

    def accumulate(ci, b):
        def sub_body(r, _):
            def acc_body(j, av):
                d = r * SUBR + j * 16
                return av + gat_b[b][pl.ds(d, 16)] * td_b[b][pl.ds(d, 16)]
            av = lax.fori_loop(0, JV, acc_body, jnp.zeros((16,), jnp.float32),
                               unroll=8)
            acc_v[pl.ds((ci * SR_PER_CH + r) * 16, 16)] = av
            return 0
        lax.fori_loop(0, SR_PER_CH, sub_body, 0)

    load_and_index(0, 0)
    fire(0)
    for ci in range(1, N_CHUNKS):
        b = ci % 2
        load_and_index(ci, b)
        drain(1 - b)
        fire(b)
        accumulate(ci - 1, 1 - b)
    drain((N_CHUNKS - 1) % 2)
    accumulate(N_CHUNKS - 1, (N_CHUNKS - 1) % 2)
    pltpu.sync_copy(acc_v, out_hbm.at[wid])


def _tc_body(lp_ref, act_ref, td_ref, out_ref):
    for p in range(TC_BLK):
        act = act_ref[p]
        td = td_ref[p]
        acc = jnp.zeros((32, 128), jnp.float32)
        for v4 in range(4):
            for v8 in range(8):
                acc = acc + jnp.where(act == v4 * 8 + v8,
                                      lp_ref[p, v4, :, v8, :], 0.0)
        out_ref[p] = acc * td


_tc_select = pl.pallas_call(
    _tc_body,
    grid=(TC_ROWS // TC_BLK,),
    in_specs=[
        pl.BlockSpec((TC_BLK, 4, 32, 8, 128),
                     lambda r: (SC_ROWS // TC_BLK + r, 0, 0, 0, 0)),
        pl.BlockSpec((TC_BLK, 32, 128), lambda r: (SC_ROWS // TC_BLK + r, 0, 0)),
        pl.BlockSpec((TC_BLK, 32, 128), lambda r: (SC_ROWS // TC_BLK + r, 0, 0)),
    ],
    out_specs=pl.BlockSpec((TC_BLK, 32, 128), lambda r: (r, 0, 0)),
    out_shape=jax.ShapeDtypeStruct((TC_ROWS, 32, 128), jnp.float32),
)


def kernel(log_policies, td_errors, actions):
    lp6 = jnp.transpose(
        jnp.transpose(log_policies, (0, 2, 3, 1))
        .reshape(A, T, V // 8, 8, BS // 128, 128),
        (0, 1, 2, 4, 3, 5))
    lp_flat = lp6.reshape(-1)
    lp_rows = lp6.reshape(NROWS, V // 8, BS // 128, 8, 128)
    act_flat = jnp.transpose(actions.astype(jnp.int32), (0, 2, 3, 1)).reshape(-1)
    td_flat = jnp.transpose(td_errors.astype(jnp.float32), (0, 2, 3, 1)).reshape(-1)
    act_rows = act_flat.reshape(NROWS, BS // 128, 128)
    td_rows = td_flat.reshape(NROWS, BS // 128, 128)

    sc_partials = _sc_gather_reduce(lp_flat, act_flat, td_flat)
    tc_partials = _tc_select(lp_rows, act_rows, td_rows)

    sc_rows = sc_partials.reshape(SC_ROWS, 4 * 16).sum(axis=-1)
    tc_rows = tc_partials.reshape(TC_ROWS, 32 * 128).sum(axis=-1)
    rows = jnp.concatenate([sc_rows, tc_rows])
    return rows.reshape(A, T) * (-1.0 / BS)


# speedup vs baseline: 1.5083x; 1.0191x over previous
import functools

import jax
import jax.numpy as jnp
from jax import lax
from jax.experimental import pallas as pl
from jax.experimental.pallas import tpu as pltpu
from jax.experimental.pallas import tpu_sc as plsc

A, BS, T, V = 8, 4096, 50, 32
NROWS = A * T
NC, NS = 2, 16
NW = NC * NS
SUBR = 1024

TC_ROWS = 112
SC_ROWS = NROWS - TC_ROWS
SR_PER_W = SC_ROWS * 4 // NW
PER_W = SUBR * SR_PER_W
SR_PER_CH = 9
CH = SUBR * SR_PER_CH
N_CHUNKS = SR_PER_W // SR_PER_CH
TC_BLK = 4
JV = SUBR // 16

_mesh = plsc.VectorSubcoreMesh(core_axis_name="c", subcore_axis_name="s")


@functools.partial(
    pl.kernel,
    mesh=_mesh,
    out_type=jax.ShapeDtypeStruct((NW, SR_PER_W * 16), jnp.float32),
    compiler_params=pltpu.CompilerParams(needs_layout_passes=False),
    scratch_types=[
        pltpu.VMEM((CH,), jnp.int32),
        pltpu.VMEM((CH,), jnp.int32),
        pltpu.VMEM((CH,), jnp.float32),
        pltpu.VMEM((CH,), jnp.float32),
        pltpu.VMEM((CH,), jnp.int32),
        pltpu.VMEM((CH,), jnp.int32),
        pltpu.VMEM((CH,), jnp.float32),
        pltpu.VMEM((CH,), jnp.float32),
        pltpu.VMEM((SR_PER_W * 16,), jnp.float32),
        pltpu.SemaphoreType.DMA,
        pltpu.SemaphoreType.DMA,
    ],
)
def _sc_gather_reduce(lp_hbm, act_hbm, td_hbm, out_hbm,
                      act0, act1, td0, td1, idx0, idx1, gat0, gat1,
                      acc_v, sem_in, sem_g):
    c = lax.axis_index("c")
    s = lax.axis_index("s")
    wid = s * NC + c
    pbase = wid * PER_W
    g0 = wid * SR_PER_W
    lanes = lax.iota(jnp.int32, 16)
    act_b, td_b, idx_b, gat_b = (act0, act1), (td0, td1), (idx0, idx1), (gat0, gat1)

    def load_and_index(ci, b):
        cb = pbase + ci * CH
        pltpu.async_copy(act_hbm.at[pl.ds(cb, CH)], act_b[b], sem_in)
        pltpu.async_copy(td_hbm.at[pl.ds(cb, CH)], td_b[b], sem_in).wait()
        pltpu.make_async_copy(act_hbm.at[pl.ds(cb, CH)], act_b[b],
                              sem_in).wait()

        def sub_body(r, _):
            g = g0 + ci * SR_PER_CH + r
            plane = (g >> 2) * (V * BS)
            b0 = (g & 3) << 10

            def idx_body(j, _):
                bj = b0 + j * 16
                sb = plane + ((bj >> 7) << 10) + (bj & 127)
                av = act_b[b][pl.ds(r * SUBR + j * 16, 16)]
                idx_b[b][pl.ds(r * SUBR + j * 16, 16)] = (
                    (sb + lanes) + ((av >> 3) << 15) + ((av & 7) << 7))
                return 0
            lax.fori_loop(0, JV, idx_body, 0, unroll=8)
            return 0
        lax.fori_loop(0, SR_PER_CH, sub_body, 0)

    def fire(b):
        pltpu.async_copy(lp_hbm.at[idx_b[b]], gat_b[b], sem_g)

    def drain(b):
        pltpu.make_async_copy(lp_hbm.at[idx_b[b]], gat_b[b], sem_g).wait()

    def accumulate(ci, b):
        def sub_body(r, _):
            def acc_body(j, av):
                d = r * SUBR + j * 16
                return av + gat_b[b][pl.ds(d, 16)] * td_b[b][pl.ds(d, 16)]
            av = lax.fori_loop(0, JV, acc_body, jnp.zeros((16,), jnp.float32),
                               unroll=8)
            acc_v[pl.ds((ci * SR_PER_CH + r) * 16, 16)] = av
            return 0
        lax.fori_loop(0, SR_PER_CH, sub_body, 0)

    load_and_index(0, 0)
    fire(0)
    for ci in range(1, N_CHUNKS):
        b = ci % 2
        load_and_index(ci, b)
        drain(1 - b)
        fire(b)
        accumulate(ci - 1, 1 - b)
    drain((N_CHUNKS - 1) % 2)
    accumulate(N_CHUNKS - 1, (N_CHUNKS - 1) % 2)
    pltpu.sync_copy(acc_v, out_hbm.at[wid])


def _tc_body(lp_ref, act_ref, td_ref, out_ref):
    for p in range(TC_BLK):
        act = act_ref[p]
        td = td_ref[p]
        acc = jnp.zeros((32, 128), jnp.float32)
        for v4 in range(4):
            for v8 in range(8):
                acc = acc + jnp.where(act == v4 * 8 + v8,
                                      lp_ref[p, v4, :, v8, :], 0.0)
        out_ref[p] = acc * td


_tc_select = pl.pallas_call(
    _tc_body,
    grid=(TC_ROWS // TC_BLK,),
    in_specs=[
        pl.BlockSpec((TC_BLK, 4, 32, 8, 128),
                     lambda r: (SC_ROWS // TC_BLK + r, 0, 0, 0, 0)),
        pl.BlockSpec((TC_BLK, 32, 128), lambda r: (SC_ROWS // TC_BLK + r, 0, 0)),
        pl.BlockSpec((TC_BLK, 32, 128), lambda r: (SC_ROWS // TC_BLK + r, 0, 0)),
    ],
    out_specs=pl.BlockSpec((TC_BLK, 32, 128), lambda r: (r, 0, 0)),
    out_shape=jax.ShapeDtypeStruct((TC_ROWS, 32, 128), jnp.float32),
)


def kernel(log_policies, td_errors, actions):
    lp6 = jnp.transpose(
        jnp.transpose(log_policies, (0, 2, 3, 1))
        .reshape(A, T, V // 8, 8, BS // 128, 128),
        (0, 1, 2, 4, 3, 5))
    lp_flat = lp6.reshape(-1)
    lp_rows = lp6.reshape(NROWS, V // 8, BS // 128, 8, 128)
    act_flat = jnp.transpose(actions.astype(jnp.int32), (0, 2, 3, 1)).reshape(-1)
    td_flat = jnp.transpose(td_errors.astype(jnp.float32), (0, 2, 3, 1)).reshape(-1)
    act_rows = act_flat.reshape(NROWS, BS // 128, 128)
    td_rows = td_flat.reshape(NROWS, BS // 128, 128)

    sc_partials = _sc_gather_reduce(lp_flat, act_flat, td_flat)
    tc_partials = _tc_select(lp_rows, act_rows, td_rows)

    sc_rows = sc_partials.reshape(SC_ROWS, 4 * 16).sum(axis=-1)
    tc_rows = tc_partials.reshape(TC_ROWS, 32 * 128).sum(axis=-1)
    rows = jnp.concatenate([sc_rows, tc_rows])
    return rows.reshape(A, T) * (-1.0 / BS)
